# baseline (speedup 1.0000x reference)
import functools

import jax
import jax.numpy as jnp
from jax import lax
from jax.experimental import pallas as pl
from jax.experimental.pallas import tpu as pltpu
from jax.experimental.pallas import tpu_sc as plsc

B = 4096
K = 20
D = 128
DE = 16
DT = 128
H = 2
DH = 64
B3 = 3 * B
NK = B3 * K
NE = 1600000

NC = 2
NS = 16
NW = NC * NS

QPW = B3 // NW
NPW = NK // NW
CH = 128
NCH_Q = QPW // CH
CPS = QPW // CH
NCH_N = K * CPS
NCH_E = K * CPS

NB = 128
GRID = B3 // NB


def _sc_gather(node_features, edge_features, idx_q, idx_n, idx_e):
  mesh = plsc.VectorSubcoreMesh(core_axis_name="c", subcore_axis_name="s")

  @functools.partial(
      pl.kernel,
      out_type=[
          jax.ShapeDtypeStruct((B3, D), jnp.float32),
          jax.ShapeDtypeStruct((K, B3, D), jnp.float32),
          jax.ShapeDtypeStruct((K, B3, D), jnp.float32),
      ],
      mesh=mesh,
      compiler_params=pltpu.CompilerParams(use_tc_tiling_on_sc=False),
      scratch_types=[
          pltpu.VMEM((QPW,), jnp.int32),
          pltpu.VMEM((NPW,), jnp.int32),
          pltpu.VMEM((NPW,), jnp.int32),
          pltpu.VMEM((CH, D), jnp.float32),
          pltpu.VMEM((CH, D), jnp.float32),
          pltpu.VMEM((CH, D), jnp.float32),
          pltpu.VMEM((CH, D), jnp.float32),
          pltpu.SemaphoreType.DMA,
          pltpu.SemaphoreType.DMA,
      ],
  )
  def k(ntab, etab, idxq, idxn, idxe, outq, outn, oute,
        idxq_v, idxn_v, idxe_v, nb0, nb1, eb0, eb1, s0, s1):
    wid = lax.axis_index("s") * NC + lax.axis_index("c")

    base_q = pl.multiple_of(wid * QPW, 8)
    pltpu.sync_copy(idxq.at[pl.ds(base_q, QPW)], idxq_v)

    def stage(j, carry):
      src = pl.multiple_of(j * B3 + base_q, 8)
      dst = pl.multiple_of(j * QPW, 8)
      pltpu.sync_copy(idxn.at[pl.ds(src, QPW)], idxn_v.at[pl.ds(dst, QPW)])
      pltpu.sync_copy(idxe.at[pl.ds(src, QPW)], idxe_v.at[pl.ds(dst, QPW)])
      return carry

    lax.fori_loop(0, K, stage, 0)

    for c in range(NCH_Q):
      pltpu.async_copy(ntab.at[idxq_v.at[pl.ds(c * CH, CH)]], nb0, s0).wait()
      pltpu.sync_copy(nb0, outq.at[pl.ds(base_q + c * CH, CH)])

    nbufs = (nb0, nb1)
    ebufs = (eb0, eb1)
    sems = (s0, s1)

    def n_dst(c):
      slab = c // CPS
      row = pl.multiple_of(base_q + (c % CPS) * CH, 8)
      return outn.at[slab, pl.ds(row, CH)]

    def e_dst(c):
      slab = c // CPS
      row = pl.multiple_of(base_q + (c % CPS) * CH, 8)
      return oute.at[slab, pl.ds(row, CH)]

    pltpu.async_copy(ntab.at[idxn_v.at[pl.ds(0, CH)]], nb0, s0)
    pltpu.async_copy(ntab.at[idxn_v.at[pl.ds(CH, CH)]], nb1, s1)

    def nbody(i, carry):
      for t in range(2):
        c = 2 * i + t
        buf, sem = nbufs[t], sems[t]
        pltpu.make_async_copy(ntab.at[pl.ds(0, CH)], buf, sem).wait()
        pltpu.sync_copy(buf, n_dst(c))

        @pl.when(c + 2 < NCH_N)
        def _():
          off2 = pl.multiple_of((c + 2) * CH, 8)
          pltpu.async_copy(ntab.at[idxn_v.at[pl.ds(off2, CH)]], buf, sem)
      return carry

    lax.fori_loop(0, NCH_N // 2, nbody, 0)

    pltpu.async_copy(etab.at[idxe_v.at[pl.ds(0, CH)]], eb0, s0)
    pltpu.async_copy(etab.at[idxe_v.at[pl.ds(CH, CH)]], eb1, s1)

    def ebody(i, carry):
      for t in range(2):
        c = 2 * i + t
        buf, sem = ebufs[t], sems[t]
        pltpu.make_async_copy(etab.at[pl.ds(0, CH)], buf, sem).wait()
        pltpu.sync_copy(buf, e_dst(c))

        @pl.when(c + 2 < NCH_E)
        def _():
          off2 = pl.multiple_of((c + 2) * CH, 8)
          pltpu.async_copy(etab.at[idxe_v.at[pl.ds(off2, CH)]], buf, sem)
      return carry

    lax.fori_loop(0, NCH_E // 2, ebody, 0)

  return k(node_features, edge_features, idx_q, idx_n, idx_e)


def _fast_cos(x):
  n = jnp.round(x * 0.15915494309189535)
  r = x - n * 6.28125 - n * 1.9353071795864769e-03
  r2 = r * r
  p = jnp.float32(1.736913401585966e-09)
  for c in (-2.711337329987122e-07, 2.47734242079983e-05,
            -0.0013887970411328634, 0.041666524363789405,
            -0.4999999177196379, 0.9999999922771011):
    p = p * r2 + jnp.float32(c)
  return p


def _tc_body(nf_ref, ngh_ref, eg_ref, r8_ref, dtc_ref,
             wq0_ref, wq1_ref, wk0_ref, wke_ref, wkt_ref,
             wv0_ref, wve_ref, wvt_ref,
             w1a_ref, w1b_ref, b1_ref, w2_ref, b2_ref,
             tw_ref, tb_ref, out_ref):
  f32 = jnp.float32
  nf = nf_ref[...]
  ngh = ngh_ref[...].reshape(K * NB, D)
  eg128 = eg_ref[...].reshape(K * NB, D)
  r8 = r8_ref[...].reshape(K * NB, 1)
  dtc = dtc_ref[...].reshape(K * NB, 1)
  lanegrp = lax.broadcasted_iota(jnp.int32, (1, D), 1) // DE
  masked_eg = jnp.where(lanegrp == r8, eg128, 0.0)
  tw = tw_ref[...]
  tb = tb_ref[...]

  ktime = _fast_cos(dtc * tw + tb)
  qtime = _fast_cos(tb)

  dot = functools.partial(jnp.dot, preferred_element_type=f32)
  q = dot(nf, wq0_ref[...]) + dot(qtime, wq1_ref[...])
  kk = (dot(ngh, wk0_ref[...]) + dot(masked_eg, wke_ref[...])
        + dot(ktime, wkt_ref[...]))
  vv = (dot(ngh, wv0_ref[...]) + dot(masked_eg, wve_ref[...])
        + dot(ktime, wvt_ref[...]))

  scale = 1.0 / (DH ** 0.5)
  lane_h = lax.broadcasted_iota(jnp.int32, (D, H), 0) // DH
  col_h = lax.broadcasted_iota(jnp.int32, (D, H), 1)
  eh = jnp.where(lane_h == col_h, scale, 0.0).astype(f32)
  row_h = lax.broadcasted_iota(jnp.int32, (H, D), 0)
  colD_h = lax.broadcasted_iota(jnp.int32, (H, D), 1) // DH
  fh = jnp.where(row_h == colD_h, 1.0, 0.0).astype(f32)

  qt = jnp.concatenate([q] * K, axis=0)
  s2 = dot(kk * qt, eh)
  s3 = s2.reshape(K, NB, H)
  m = jnp.max(s3, axis=0, keepdims=True)
  e = jnp.exp(s3 - m)
  attn = e / jnp.sum(e, axis=0, keepdims=True)

  att_out = jnp.zeros((NB, H * DH), dtype=f32)
  for k in range(K):
    bc = dot(attn[k], fh)
    att_out = att_out + bc * vv[k * NB:(k + 1) * NB, :]

  hidden = jnp.maximum(
      dot(att_out, w1a_ref[...]) + dot(nf, w1b_ref[...]) + b1_ref[...], 0.0)
  out_ref[...] = dot(hidden, w2_ref[...]) + b2_ref[...]


def _tc_dense(node_feat, neigh, eg, r8col, dtcol,
              Wq0, Wq1, Wk0, Wke, Wkt, Wv0, Wve, Wvt,
              W1a, W1b, b1, W2, b2, tw, tb):
  def full(shape):
    return pl.BlockSpec(shape, lambda i: (0,) * len(shape))

  return pl.pallas_call(
      _tc_body,
      grid=(GRID,),
      in_specs=[
          pl.BlockSpec((NB, D), lambda i: (i, 0)),
          pl.BlockSpec((K, NB, D), lambda i: (0, i, 0)),
          pl.BlockSpec((K, NB, D), lambda i: (0, i, 0)),
          pl.BlockSpec((K, NB, 1), lambda i: (0, i, 0)),
          pl.BlockSpec((K, NB, 1), lambda i: (0, i, 0)),
          full((D, H * DH)), full((DT, H * DH)),
          full((D, H * DH)), full((D, H * DH)), full((DT, H * DH)),
          full((D, H * DH)), full((D, H * DH)), full((DT, H * DH)),
          full((H * DH, D)), full((D, D)), full((1, D)),
          full((D, D)), full((1, D)),
          full((1, DT)), full((1, DT)),
      ],
      out_specs=pl.BlockSpec((NB, D), lambda i: (i, 0)),
      out_shape=jax.ShapeDtypeStruct((B3, D), jnp.float32),
  )(node_feat, neigh, eg, r8col, dtcol,
    Wq0, Wq1, Wk0, Wke, Wkt, Wv0, Wve, Wvt,
    W1a, W1b, b1, W2, b2, tw, tb)


def kernel(source_nodes, destination_nodes, negative_nodes, edge_times,
           edge_idxs, neighbor_idx, neighbor_edge_idx, neighbor_times,
           node_features, edge_features, time_w, time_b,
           Wq, Wk, Wv, W1, b1, W2, b2):
  del edge_idxs
  nodes = jnp.concatenate(
      [source_nodes, destination_nodes, negative_nodes]).astype(jnp.int32)
  times3 = jnp.concatenate([edge_times, edge_times, edge_times])

  idx_n = neighbor_idx.T.reshape(-1).astype(jnp.int32)
  eidx_t = neighbor_edge_idx.T.astype(jnp.int32)
  idx_e = (eidx_t >> 3).reshape(-1)
  r8col = (eidx_t & 7).reshape(K, B3, 1)
  dt_t = times3[None, :] - neighbor_times.T
  dtcol = dt_t.reshape(K, B3, 1)

  ef128 = edge_features.reshape(NE * DE // 128, 128)
  q_feat, neigh, eg = _sc_gather(
      node_features, ef128, nodes, idx_n, idx_e)

  h = _tc_dense(
      q_feat, neigh, eg, r8col, dtcol,
      Wq[:D], Wq[D:],
      Wk[:D], jnp.tile(Wk[D:D + DE], (D // DE, 1)), Wk[D + DE:],
      Wv[:D], jnp.tile(Wv[D:D + DE], (D // DE, 1)), Wv[D + DE:],
      W1[:H * DH], W1[H * DH:], b1.reshape(1, D),
      W2, b2.reshape(1, D),
      time_w.reshape(1, DT), time_b.reshape(1, DT))

  return h[:B], h[B:2 * B], h[2 * B:]


# speedup vs baseline: 1.0727x; 1.0727x over previous
import functools

import jax
import jax.numpy as jnp
from jax import lax
from jax.experimental import pallas as pl
from jax.experimental.pallas import tpu as pltpu
from jax.experimental.pallas import tpu_sc as plsc

B = 4096
K = 20
D = 128
DE = 16
DT = 128
H = 2
DH = 64
B3 = 3 * B
NK = B3 * K
NE = 1600000

NC = 2
NS = 16
NW = NC * NS

QPW = B3 // NW
NPW = NK // NW
CH = 128
CPS = QPW // CH
NCH = K * CPS

NB = 128
GRID = B3 // NB

_MESH = plsc.VectorSubcoreMesh(core_axis_name="c", subcore_axis_name="s")
_SC_PARAMS = pltpu.CompilerParams(use_tc_tiling_on_sc=False)


def _stage_indices(idx_hbm, idx_v, base):
  def stage(j, carry):
    src = pl.multiple_of(j * B3 + base, 8)
    dst = pl.multiple_of(j * QPW, 8)
    pltpu.sync_copy(idx_hbm.at[pl.ds(src, QPW)], idx_v.at[pl.ds(dst, QPW)])
    return carry

  lax.fori_loop(0, K, stage, 0)


def _slab_pipeline(tab, idx_v, out3d, bufs, sems, base):
  def dst(c):
    slab = c // CPS
    row = pl.multiple_of(base + (c % CPS) * CH, 8)
    return out3d.at[slab, pl.ds(row, CH)]

  pltpu.async_copy(tab.at[idx_v.at[pl.ds(0, CH)]], bufs[0], sems[0])
  pltpu.async_copy(tab.at[idx_v.at[pl.ds(CH, CH)]], bufs[1], sems[1])

  def body(i, carry):
    for t in range(2):
      c = 2 * i + t
      buf, sem = bufs[t], sems[t]
      pltpu.make_async_copy(tab.at[pl.ds(0, CH)], buf, sem).wait()
      pltpu.sync_copy(buf, dst(c))

      @pl.when(c + 2 < NCH)
      def _():
        off2 = pl.multiple_of((c + 2) * CH, 8)
        pltpu.async_copy(tab.at[idx_v.at[pl.ds(off2, CH)]], buf, sem)
    return carry

  lax.fori_loop(0, NCH // 2, body, 0)


def _sc_gather_nodes(node_features, idx_q, idx_n):
  @functools.partial(
      pl.kernel,
      out_type=[
          jax.ShapeDtypeStruct((B3, D), jnp.float32),
          jax.ShapeDtypeStruct((K, B3, D), jnp.float32),
      ],
      mesh=_MESH,
      compiler_params=_SC_PARAMS,
      scratch_types=[
          pltpu.VMEM((QPW,), jnp.int32),
          pltpu.VMEM((NPW,), jnp.int32),
          pltpu.VMEM((CH, D), jnp.float32),
          pltpu.VMEM((CH, D), jnp.float32),
          pltpu.SemaphoreType.DMA,
          pltpu.SemaphoreType.DMA,
      ],
  )
  def k(ntab, idxq, idxn, outq, outn, idxq_v, idxn_v, b0, b1, s0, s1):
    wid = lax.axis_index("s") * NC + lax.axis_index("c")
    base = pl.multiple_of(wid * QPW, 8)

    pltpu.sync_copy(idxq.at[pl.ds(base, QPW)], idxq_v)
    _stage_indices(idxn, idxn_v, base)

    for c in range(CPS):
      pltpu.async_copy(ntab.at[idxq_v.at[pl.ds(c * CH, CH)]], b0, s0).wait()
      pltpu.sync_copy(b0, outq.at[pl.ds(base + c * CH, CH)])

    _slab_pipeline(ntab, idxn_v, outn, (b0, b1), (s0, s1), base)

  return k(node_features, idx_q, idx_n)


def _sc_gather_edges(edge_features, idx_e):
  @functools.partial(
      pl.kernel,
      out_type=jax.ShapeDtypeStruct((K, B3, DE), jnp.float32),
      mesh=_MESH,
      compiler_params=_SC_PARAMS,
      scratch_types=[
          pltpu.VMEM((NPW,), jnp.int32),
          pltpu.VMEM((CH, DE), jnp.float32),
          pltpu.VMEM((CH, DE), jnp.float32),
          pltpu.SemaphoreType.DMA,
          pltpu.SemaphoreType.DMA,
      ],
  )
  def k(etab, idxe, oute, idxe_v, b0, b1, s0, s1):
    wid = lax.axis_index("s") * NC + lax.axis_index("c")
    base = pl.multiple_of(wid * QPW, 8)
    _stage_indices(idxe, idxe_v, base)
    _slab_pipeline(etab, idxe_v, oute, (b0, b1), (s0, s1), base)

  return k(edge_features, idx_e)


def _fast_cos(x):
  n = jnp.round(x * 0.15915494309189535)
  r = x - n * 6.28125 - n * 1.9353071795864769e-03
  r2 = r * r
  p = jnp.float32(1.736913401585966e-09)
  for c in (-2.711337329987122e-07, 2.47734242079983e-05,
            -0.0013887970411328634, 0.041666524363789405,
            -0.4999999177196379, 0.9999999922771011):
    p = p * r2 + jnp.float32(c)
  return p


def _tc_body(nf_ref, ngh_ref, eg_ref, dtc_ref,
             wq0_ref, wq1_ref, wk0_ref, wke_ref, wkt_ref,
             wv0_ref, wve_ref, wvt_ref,
             w1a_ref, w1b_ref, b1_ref, w2_ref, b2_ref,
             tw_ref, tb_ref, out_ref):
  f32 = jnp.float32
  nf = nf_ref[...]
  ngh = ngh_ref[...].reshape(K * NB, D)
  eg = eg_ref[...].reshape(K * NB, DE)
  dtc = dtc_ref[...].reshape(K * NB, 1)
  tw = tw_ref[...]
  tb = tb_ref[...]

  ktime = _fast_cos(dtc * tw + tb)
  qtime = _fast_cos(tb)

  dot = functools.partial(jnp.dot, preferred_element_type=f32)
  q = dot(nf, wq0_ref[...]) + dot(qtime, wq1_ref[...])
  kk = (dot(ngh, wk0_ref[...]) + dot(eg, wke_ref[...])
        + dot(ktime, wkt_ref[...]))
  vv = (dot(ngh, wv0_ref[...]) + dot(eg, wve_ref[...])
        + dot(ktime, wvt_ref[...]))

  scale = 1.0 / (DH ** 0.5)
  lane_h = lax.broadcasted_iota(jnp.int32, (D, H), 0) // DH
  col_h = lax.broadcasted_iota(jnp.int32, (D, H), 1)
  eh = jnp.where(lane_h == col_h, scale, 0.0).astype(f32)
  row_h = lax.broadcasted_iota(jnp.int32, (H, D), 0)
  colD_h = lax.broadcasted_iota(jnp.int32, (H, D), 1) // DH
  fh = jnp.where(row_h == colD_h, 1.0, 0.0).astype(f32)

  qt = jnp.concatenate([q] * K, axis=0)
  s2 = dot(kk * qt, eh)
  s3 = s2.reshape(K, NB, H)
  m = jnp.max(s3, axis=0, keepdims=True)
  e = jnp.exp(s3 - m)
  attn = e / jnp.sum(e, axis=0, keepdims=True)

  att_out = jnp.zeros((NB, H * DH), dtype=f32)
  for k in range(K):
    bc = dot(attn[k], fh)
    att_out = att_out + bc * vv[k * NB:(k + 1) * NB, :]

  hidden = jnp.maximum(
      dot(att_out, w1a_ref[...]) + dot(nf, w1b_ref[...]) + b1_ref[...], 0.0)
  out_ref[...] = dot(hidden, w2_ref[...]) + b2_ref[...]


def _tc_dense(node_feat, neigh, eg, dtcol,
              Wq0, Wq1, Wk0, Wke, Wkt, Wv0, Wve, Wvt,
              W1a, W1b, b1, W2, b2, tw, tb):
  def full(shape):
    return pl.BlockSpec(shape, lambda i: (0,) * len(shape))

  return pl.pallas_call(
      _tc_body,
      grid=(GRID,),
      in_specs=[
          pl.BlockSpec((NB, D), lambda i: (i, 0)),
          pl.BlockSpec((K, NB, D), lambda i: (0, i, 0)),
          pl.BlockSpec((K, NB, DE), lambda i: (0, i, 0)),
          pl.BlockSpec((K, NB, 1), lambda i: (0, i, 0)),
          full((D, H * DH)), full((DT, H * DH)),
          full((D, H * DH)), full((DE, H * DH)), full((DT, H * DH)),
          full((D, H * DH)), full((DE, H * DH)), full((DT, H * DH)),
          full((H * DH, D)), full((D, D)), full((1, D)),
          full((D, D)), full((1, D)),
          full((1, DT)), full((1, DT)),
      ],
      out_specs=pl.BlockSpec((NB, D), lambda i: (i, 0)),
      out_shape=jax.ShapeDtypeStruct((B3, D), jnp.float32),
  )(node_feat, neigh, eg, dtcol,
    Wq0, Wq1, Wk0, Wke, Wkt, Wv0, Wve, Wvt,
    W1a, W1b, b1, W2, b2, tw, tb)


def kernel(source_nodes, destination_nodes, negative_nodes, edge_times,
           edge_idxs, neighbor_idx, neighbor_edge_idx, neighbor_times,
           node_features, edge_features, time_w, time_b,
           Wq, Wk, Wv, W1, b1, W2, b2):
  del edge_idxs
  nodes = jnp.concatenate(
      [source_nodes, destination_nodes, negative_nodes]).astype(jnp.int32)
  times3 = jnp.concatenate([edge_times, edge_times, edge_times])

  idx_n = neighbor_idx.T.reshape(-1).astype(jnp.int32)
  idx_e = neighbor_edge_idx.T.reshape(-1).astype(jnp.int32)
  dt_t = times3[None, :] - neighbor_times.T
  dtcol = dt_t.reshape(K, B3, 1)

  eg = _sc_gather_edges(edge_features, idx_e)
  q_feat, neigh = _sc_gather_nodes(node_features, nodes, idx_n)

  h = _tc_dense(
      q_feat, neigh, eg, dtcol,
      Wq[:D], Wq[D:],
      Wk[:D], Wk[D:D + DE], Wk[D + DE:],
      Wv[:D], Wv[D:D + DE], Wv[D + DE:],
      W1[:H * DH], W1[H * DH:], b1.reshape(1, D),
      W2, b2.reshape(1, D),
      time_w.reshape(1, DT), time_b.reshape(1, DT))

  return h[:B], h[B:2 * B], h[2 * B:]


# speedup vs baseline: 1.0858x; 1.0121x over previous
import functools

import jax
import jax.numpy as jnp
from jax import lax
from jax.experimental import pallas as pl
from jax.experimental.pallas import tpu as pltpu
from jax.experimental.pallas import tpu_sc as plsc

B = 4096
K = 20
D = 128
DE = 16
DT = 128
H = 2
DH = 64
B3 = 3 * B
NK = B3 * K
NE = 1600000

NC = 2
NS = 16
NW = NC * NS

QPW = B3 // NW
NPW = NK // NW
CH = 128
CPS = QPW // CH
NCH = K * CPS

NB = 256
GRID = B3 // NB

_MESH = plsc.VectorSubcoreMesh(core_axis_name="c", subcore_axis_name="s",
                               num_cores=NC, num_subcores=NS)
_SC_PARAMS = pltpu.CompilerParams(use_tc_tiling_on_sc=False)


def _stage_indices(idx_hbm, idx_v, base):
  def stage(j, carry):
    src = pl.multiple_of(j * B3 + base, 8)
    dst = pl.multiple_of(j * QPW, 8)
    pltpu.sync_copy(idx_hbm.at[pl.ds(src, QPW)], idx_v.at[pl.ds(dst, QPW)])
    return carry

  lax.fori_loop(0, K, stage, 0)


def _slab_pipeline(tab, idx_v, out3d, bufs, sems, base):
  def dst(c):
    slab = c // CPS
    row = pl.multiple_of(base + (c % CPS) * CH, 8)
    return out3d.at[slab, pl.ds(row, CH)]

  pltpu.async_copy(tab.at[idx_v.at[pl.ds(0, CH)]], bufs[0], sems[0])
  pltpu.async_copy(tab.at[idx_v.at[pl.ds(CH, CH)]], bufs[1], sems[1])

  def body(i, carry):
    for t in range(2):
      c = 2 * i + t
      buf, sem = bufs[t], sems[t]
      pltpu.make_async_copy(tab.at[pl.ds(0, CH)], buf, sem).wait()
      pltpu.sync_copy(buf, dst(c))

      @pl.when(c + 2 < NCH)
      def _():
        off2 = pl.multiple_of((c + 2) * CH, 8)
        pltpu.async_copy(tab.at[idx_v.at[pl.ds(off2, CH)]], buf, sem)
    return carry

  lax.fori_loop(0, NCH // 2, body, 0)


def _sc_gather_nodes(node_features, idx_q, idx_n):
  @functools.partial(
      pl.kernel,
      out_type=[
          jax.ShapeDtypeStruct((B3, D), jnp.float32),
          jax.ShapeDtypeStruct((K, B3, D), jnp.float32),
      ],
      mesh=_MESH,
      compiler_params=_SC_PARAMS,
      scratch_types=[
          pltpu.VMEM((QPW,), jnp.int32),
          pltpu.VMEM((NPW,), jnp.int32),
          pltpu.VMEM((CH, D), jnp.float32),
          pltpu.VMEM((CH, D), jnp.float32),
          pltpu.SemaphoreType.DMA,
          pltpu.SemaphoreType.DMA,
      ],
  )
  def k(ntab, idxq, idxn, outq, outn, idxq_v, idxn_v, b0, b1, s0, s1):
    wid = lax.axis_index("s") * NC + lax.axis_index("c")
    base = pl.multiple_of(wid * QPW, 8)

    pltpu.sync_copy(idxq.at[pl.ds(base, QPW)], idxq_v)
    _stage_indices(idxn, idxn_v, base)

    for c in range(CPS):
      pltpu.async_copy(ntab.at[idxq_v.at[pl.ds(c * CH, CH)]], b0, s0).wait()
      pltpu.sync_copy(b0, outq.at[pl.ds(base + c * CH, CH)])

    _slab_pipeline(ntab, idxn_v, outn, (b0, b1), (s0, s1), base)

  return k(node_features, idx_q, idx_n)


def _sc_gather_edges(edge_features, idx_e):
  @functools.partial(
      pl.kernel,
      out_type=jax.ShapeDtypeStruct((K, B3, DE), jnp.float32),
      mesh=_MESH,
      compiler_params=_SC_PARAMS,
      scratch_types=[
          pltpu.VMEM((NPW,), jnp.int32),
          pltpu.VMEM((CH, DE), jnp.float32),
          pltpu.VMEM((CH, DE), jnp.float32),
          pltpu.SemaphoreType.DMA,
          pltpu.SemaphoreType.DMA,
      ],
  )
  def k(etab, idxe, oute, idxe_v, b0, b1, s0, s1):
    wid = lax.axis_index("s") * NC + lax.axis_index("c")
    base = pl.multiple_of(wid * QPW, 8)
    _stage_indices(idxe, idxe_v, base)
    _slab_pipeline(etab, idxe_v, oute, (b0, b1), (s0, s1), base)

  return k(edge_features, idx_e)


def _fast_cos(x):
  n = jnp.round(x * 0.15915494309189535)
  r = x - n * 6.28125 - n * 1.9353071795864769e-03
  r2 = r * r
  p = jnp.float32(1.736913401585966e-09)
  for c in (-2.711337329987122e-07, 2.47734242079983e-05,
            -0.0013887970411328634, 0.041666524363789405,
            -0.4999999177196379, 0.9999999922771011):
    p = p * r2 + jnp.float32(c)
  return p


def _tc_body(nf_ref, ngh_ref, eg_ref, dtc_ref,
             wq0_ref, wq1_ref, wk0_ref, wke_ref, wkt_ref,
             wv0_ref, wve_ref, wvt_ref,
             w1a_ref, w1b_ref, b1_ref, w2_ref, b2_ref,
             tw_ref, tb_ref, out_ref):
  f32 = jnp.float32
  nf = nf_ref[...]
  ngh = ngh_ref[...].reshape(K * NB, D)
  eg = eg_ref[...].reshape(K * NB, DE)
  dtc = dtc_ref[...].reshape(K * NB, 1)
  tw = tw_ref[...]
  tb = tb_ref[...]

  ktime = _fast_cos(dtc * tw + tb)
  qtime = _fast_cos(tb)

  dot = functools.partial(jnp.dot, preferred_element_type=f32)
  q = dot(nf, wq0_ref[...]) + dot(qtime, wq1_ref[...])
  kk = (dot(ngh, wk0_ref[...]) + dot(eg, wke_ref[...])
        + dot(ktime, wkt_ref[...]))
  vv = (dot(ngh, wv0_ref[...]) + dot(eg, wve_ref[...])
        + dot(ktime, wvt_ref[...]))

  scale = 1.0 / (DH ** 0.5)
  lane_h = lax.broadcasted_iota(jnp.int32, (D, H), 0) // DH
  col_h = lax.broadcasted_iota(jnp.int32, (D, H), 1)
  eh = jnp.where(lane_h == col_h, scale, 0.0).astype(f32)
  row_h = lax.broadcasted_iota(jnp.int32, (H, D), 0)
  colD_h = lax.broadcasted_iota(jnp.int32, (H, D), 1) // DH
  fh = jnp.where(row_h == colD_h, 1.0, 0.0).astype(f32)

  qt = jnp.concatenate([q] * K, axis=0)
  s2 = dot(kk * qt, eh)
  s3 = s2.reshape(K, NB, H)
  m = jnp.max(s3, axis=0, keepdims=True)
  e = jnp.exp(s3 - m)
  attn = e / jnp.sum(e, axis=0, keepdims=True)

  att_out = jnp.zeros((NB, H * DH), dtype=f32)
  for k in range(K):
    bc = dot(attn[k], fh)
    att_out = att_out + bc * vv[k * NB:(k + 1) * NB, :]

  hidden = jnp.maximum(
      dot(att_out, w1a_ref[...]) + dot(nf, w1b_ref[...]) + b1_ref[...], 0.0)
  out_ref[...] = dot(hidden, w2_ref[...]) + b2_ref[...]


def _tc_dense(node_feat, neigh, eg, dtcol,
              Wq0, Wq1, Wk0, Wke, Wkt, Wv0, Wve, Wvt,
              W1a, W1b, b1, W2, b2, tw, tb):
  def full(shape):
    return pl.BlockSpec(shape, lambda i: (0,) * len(shape))

  return pl.pallas_call(
      _tc_body,
      grid=(GRID,),
      in_specs=[
          pl.BlockSpec((NB, D), lambda i: (i, 0)),
          pl.BlockSpec((K, NB, D), lambda i: (0, i, 0)),
          pl.BlockSpec((K, NB, DE), lambda i: (0, i, 0)),
          pl.BlockSpec((K, NB, 1), lambda i: (0, i, 0)),
          full((D, H * DH)), full((DT, H * DH)),
          full((D, H * DH)), full((DE, H * DH)), full((DT, H * DH)),
          full((D, H * DH)), full((DE, H * DH)), full((DT, H * DH)),
          full((H * DH, D)), full((D, D)), full((1, D)),
          full((D, D)), full((1, D)),
          full((1, DT)), full((1, DT)),
      ],
      out_specs=pl.BlockSpec((NB, D), lambda i: (i, 0)),
      out_shape=jax.ShapeDtypeStruct((B3, D), jnp.float32),
  )(node_feat, neigh, eg, dtcol,
    Wq0, Wq1, Wk0, Wke, Wkt, Wv0, Wve, Wvt,
    W1a, W1b, b1, W2, b2, tw, tb)


def kernel(source_nodes, destination_nodes, negative_nodes, edge_times,
           edge_idxs, neighbor_idx, neighbor_edge_idx, neighbor_times,
           node_features, edge_features, time_w, time_b,
           Wq, Wk, Wv, W1, b1, W2, b2):
  del edge_idxs
  nodes = jnp.concatenate(
      [source_nodes, destination_nodes, negative_nodes]).astype(jnp.int32)
  times3 = jnp.concatenate([edge_times, edge_times, edge_times])

  idx_n = neighbor_idx.T.reshape(-1).astype(jnp.int32)
  idx_e = neighbor_edge_idx.T.reshape(-1).astype(jnp.int32)
  dt_t = times3[None, :] - neighbor_times.T
  dtcol = dt_t.reshape(K, B3, 1)

  eg = _sc_gather_edges(edge_features, idx_e)
  q_feat, neigh = _sc_gather_nodes(node_features, nodes, idx_n)

  h = _tc_dense(
      q_feat, neigh, eg, dtcol,
      Wq[:D], Wq[D:],
      Wk[:D], Wk[D:D + DE], Wk[D + DE:],
      Wv[:D], Wv[D:D + DE], Wv[D + DE:],
      W1[:H * DH], W1[H * DH:], b1.reshape(1, D),
      W2, b2.reshape(1, D),
      time_w.reshape(1, DT), time_b.reshape(1, DT))

  return h[:B], h[B:2 * B], h[2 * B:]
